# baseline (speedup 1.0000x reference)
import functools

import jax
import jax.numpy as jnp
from jax import lax
from jax.experimental import pallas as pl
from jax.experimental.pallas import tpu as pltpu
from jax.experimental.pallas import tpu_sc as plsc

SEQ = 12
IMG = 32
D = IMG * IMG
BATCH = 1024

_NC, _NS = 2, 16
_NW = _NC * _NS
_B_PER_W = BATCH // _NW


@functools.lru_cache(maxsize=None)
def _make_sc_gather():
    @functools.partial(
        pl.kernel,
        mesh=plsc.VectorSubcoreMesh(core_axis_name="c", subcore_axis_name="s"),
        out_type=jax.ShapeDtypeStruct((BATCH, D), jnp.float32),
        scratch_types=[
            pltpu.VMEM((_B_PER_W,), jnp.int32),
            pltpu.VMEM((_B_PER_W, D), jnp.float32),
            pltpu.SemaphoreType.DMA,
        ],
    )
    def _sc_gather(table_hbm, idx_hbm, out_hbm, idx_v, rows_v, sem):
        wid = lax.axis_index("s") * _NC + lax.axis_index("c")
        base = wid * _B_PER_W
        pltpu.sync_copy(idx_hbm.at[pl.ds(base, _B_PER_W)], idx_v)
        pltpu.async_copy(table_hbm.at[idx_v], rows_v, sem).wait()
        pltpu.sync_copy(rows_v, out_hbm.at[pl.ds(base, _B_PER_W)])

    return _sc_gather


def _copy_x_body(x_ref, out_ref):
    out_ref[...] = x_ref[...]


def _tc_copy_x(xt):
    return pl.pallas_call(
        _copy_x_body,
        grid=(3, SEQ // 3),
        in_specs=[
            pl.BlockSpec(
                (1, 3, IMG, IMG, BATCH), lambda c, s: (c, s, 0, 0, 0)
            ),
        ],
        out_specs=pl.BlockSpec(
            (1, 3, IMG, IMG, BATCH), lambda c, s: (c, s, 0, 0, 0)
        ),
        out_shape=jax.ShapeDtypeStruct((4, SEQ, IMG, IMG, BATCH), jnp.float32),
        compiler_params=pltpu.CompilerParams(
            dimension_semantics=("arbitrary", "arbitrary"),
        ),
    )(xt)


def _emb_body(buf_hbm, emb_ref, out_ref, embt_ref):
    s = pl.program_id(0)

    @pl.when(s == 0)
    def _transpose():
        e = emb_ref[...]
        embt_ref[...] = e.T.reshape(IMG, IMG, BATCH)

    et = embt_ref[...].reshape(1, 1, IMG, IMG, BATCH)
    out_ref[...] = jnp.broadcast_to(et, (1, 6, IMG, IMG, BATCH))


def _tc_write_emb(buf, emb):
    return pl.pallas_call(
        _emb_body,
        grid=(SEQ // 6,),
        in_specs=[
            pl.BlockSpec(memory_space=pltpu.MemorySpace.HBM),
            pl.BlockSpec((BATCH, D), lambda s: (0, 0)),
        ],
        out_specs=pl.BlockSpec(
            (1, 6, IMG, IMG, BATCH), lambda s: (3, s, 0, 0, 0)
        ),
        out_shape=jax.ShapeDtypeStruct((4, SEQ, IMG, IMG, BATCH), jnp.float32),
        scratch_shapes=[pltpu.VMEM((IMG, IMG, BATCH), jnp.float32)],
        input_output_aliases={0: 0},
        compiler_params=pltpu.CompilerParams(
            dimension_semantics=("arbitrary",),
        ),
    )(buf, emb)


def kernel(x, id, table):
    xt = jnp.transpose(x, (1, 2, 3, 4, 0))
    emb = _make_sc_gather()(table, id)
    buf = _tc_copy_x(xt)
    out_t = _tc_write_emb(buf, emb)
    return jnp.transpose(out_t, (4, 0, 1, 2, 3))


# speedup vs baseline: 1.0071x; 1.0071x over previous
import functools

import jax
import jax.numpy as jnp
from jax import lax
from jax.experimental import pallas as pl
from jax.experimental.pallas import tpu as pltpu
from jax.experimental.pallas import tpu_sc as plsc

SEQ = 12
IMG = 32
D = IMG * IMG
BATCH = 1024

_NC, _NS = 2, 16
_NW = _NC * _NS
_B_PER_W = BATCH // _NW


@functools.lru_cache(maxsize=None)
def _make_sc_gather():
    @functools.partial(
        pl.kernel,
        mesh=plsc.VectorSubcoreMesh(core_axis_name="c", subcore_axis_name="s"),
        out_type=jax.ShapeDtypeStruct((BATCH, D), jnp.float32),
        scratch_types=[
            pltpu.VMEM((_B_PER_W,), jnp.int32),
            pltpu.VMEM((_B_PER_W, D), jnp.float32),
            pltpu.SemaphoreType.DMA,
        ],
    )
    def _sc_gather(table_hbm, idx_hbm, out_hbm, idx_v, rows_v, sem):
        wid = lax.axis_index("s") * _NC + lax.axis_index("c")
        base = wid * _B_PER_W
        pltpu.sync_copy(idx_hbm.at[pl.ds(base, _B_PER_W)], idx_v)
        pltpu.async_copy(table_hbm.at[idx_v], rows_v, sem).wait()
        pltpu.sync_copy(rows_v, out_hbm.at[pl.ds(base, _B_PER_W)])

    return _sc_gather


def _copy_x_body(x_ref, out_ref):
    out_ref[...] = x_ref[...]


def _tc_copy_x(xt):
    return pl.pallas_call(
        _copy_x_body,
        grid=(3, SEQ // 3),
        in_specs=[
            pl.BlockSpec(
                (1, 3, IMG, IMG, BATCH), lambda c, s: (c, s, 0, 0, 0)
            ),
        ],
        out_specs=pl.BlockSpec(
            (1, 3, IMG, IMG, BATCH), lambda c, s: (c, s, 0, 0, 0)
        ),
        out_shape=jax.ShapeDtypeStruct((4, SEQ, IMG, IMG, BATCH), jnp.float32),
        compiler_params=pltpu.CompilerParams(
            dimension_semantics=("arbitrary", "arbitrary"),
        ),
    )(xt)


def _emb_body(buf_hbm, emb_ref, out_ref, embt_ref):
    s = pl.program_id(0)

    @pl.when(s == 0)
    def _transpose():
        e = emb_ref[...]
        embt_ref[...] = e.T.reshape(IMG, IMG, BATCH)

    et = embt_ref[...].reshape(1, 1, IMG, IMG, BATCH)
    out_ref[...] = jnp.broadcast_to(et, (1, 3, IMG, IMG, BATCH))


def _tc_write_emb(buf, emb):
    return pl.pallas_call(
        _emb_body,
        grid=(SEQ // 3,),
        in_specs=[
            pl.BlockSpec(memory_space=pltpu.MemorySpace.HBM),
            pl.BlockSpec((BATCH, D), lambda s: (0, 0)),
        ],
        out_specs=pl.BlockSpec(
            (1, 3, IMG, IMG, BATCH), lambda s: (3, s, 0, 0, 0)
        ),
        out_shape=jax.ShapeDtypeStruct((4, SEQ, IMG, IMG, BATCH), jnp.float32),
        scratch_shapes=[pltpu.VMEM((IMG, IMG, BATCH), jnp.float32)],
        input_output_aliases={0: 0},
        compiler_params=pltpu.CompilerParams(
            dimension_semantics=("arbitrary",),
        ),
    )(buf, emb)


def kernel(x, id, table):
    xt = jnp.transpose(x, (1, 2, 3, 4, 0))
    emb = _make_sc_gather()(table, id)
    buf = _tc_copy_x(xt)
    out_t = _tc_write_emb(buf, emb)
    return jnp.transpose(out_t, (4, 0, 1, 2, 3))
